# Initial kernel scaffold; baseline (speedup 1.0000x reference)
import jax
import jax.numpy as jnp
from jax.experimental import pallas as pl


def kernel(x, edge_index, W, b):
    raise NotImplementedError("write your pallas kernel here")


# speedup vs baseline: 375.0015x; 375.0015x over previous
import functools

import jax
import jax.numpy as jnp
from jax import lax
from jax.experimental import pallas as pl
from jax.experimental.pallas import tpu as pltpu
from jax.experimental.pallas import tpu_sc as plsc

N_NODES = 100000
N_EDGES = 6400000
CHUNK_EDGES = 3200
NCHUNK = N_EDGES // CHUNK_EDGES
NC = 2
NS = 16
NW = NC * NS
ITERS = (NCHUNK + NW - 1) // NW

_MESH = plsc.VectorSubcoreMesh(
    core_axis_name="c", subcore_axis_name="s", num_cores=NC, num_subcores=NS
)


def _worker_id():
    c = lax.axis_index("c")
    s = lax.axis_index("s")
    return c, s, s * NC + c


@functools.partial(
    pl.kernel,
    out_type=jax.ShapeDtypeStruct((NC, N_NODES), jnp.float32),
    mesh=_MESH,
    compiler_params=pltpu.CompilerParams(needs_layout_passes=False),
    scratch_types=[
        pltpu.VMEM((CHUNK_EDGES,), jnp.int32),
        pltpu.VMEM((CHUNK_EDGES,), jnp.float32),
        pltpu.VMEM_SHARED((N_NODES,), jnp.float32),
    ],
)
def _k1_degree(ei_hbm, zeros_hbm, ones_hbm, deg_out, dstbuf, ones_v, degacc):
    c, s, w = _worker_id()

    @pl.when(s == 0)
    def _():
        pltpu.sync_copy(zeros_hbm, degacc)

    pltpu.sync_copy(ones_hbm, ones_v)
    plsc.subcore_barrier()

    def body(k, carry):
        chunk = w + NW * k

        @pl.when(chunk < NCHUNK)
        def _():
            pltpu.sync_copy(
                ei_hbm.at[1, pl.ds(chunk * CHUNK_EDGES, CHUNK_EDGES)], dstbuf
            )
            pltpu.sync_copy(ones_v, degacc.at[dstbuf], add=True)

        return carry

    lax.fori_loop(0, ITERS, body, None)
    plsc.subcore_barrier()

    @pl.when(s == 0)
    def _():
        pltpu.sync_copy(degacc, deg_out.at[c])


@functools.partial(
    pl.kernel,
    out_type=jax.ShapeDtypeStruct((NC, N_NODES), jnp.float32),
    mesh=_MESH,
    compiler_params=pltpu.CompilerParams(needs_layout_passes=False),
    scratch_types=[
        pltpu.VMEM((N_NODES,), jnp.float32),
        pltpu.VMEM((CHUNK_EDGES,), jnp.int32),
        pltpu.VMEM((CHUNK_EDGES,), jnp.int32),
        pltpu.VMEM((CHUNK_EDGES,), jnp.float32),
        pltpu.VMEM_SHARED((N_NODES,), jnp.float32),
    ],
)
def _k3_aggregate(ei_hbm, y_hbm, zeros_hbm, acc_out,
                  ybuf, srcbuf, dstbuf, msgbuf, sacc):
    c, s, w = _worker_id()

    @pl.when(s == 0)
    def _():
        pltpu.sync_copy(zeros_hbm, sacc)

    pltpu.sync_copy(y_hbm, ybuf)
    plsc.subcore_barrier()

    def body(k, carry):
        chunk = w + NW * k

        @pl.when(chunk < NCHUNK)
        def _():
            pltpu.sync_copy(
                ei_hbm.at[0, pl.ds(chunk * CHUNK_EDGES, CHUNK_EDGES)], srcbuf
            )
            pltpu.sync_copy(
                ei_hbm.at[1, pl.ds(chunk * CHUNK_EDGES, CHUNK_EDGES)], dstbuf
            )
            for t in range(CHUNK_EDGES // 16):
                idx16 = srcbuf[pl.ds(t * 16, 16)]
                msgbuf[pl.ds(t * 16, 16)] = plsc.load_gather(ybuf, [idx16])
            pltpu.sync_copy(msgbuf, sacc.at[dstbuf], add=True)

        return carry

    lax.fori_loop(0, ITERS, body, None)
    plsc.subcore_barrier()

    @pl.when(s == 0)
    def _():
        pltpu.sync_copy(sacc, acc_out.at[c])


def _k2_body(dp_ref, x_ref, y_ref, dinv_ref):
    deg = dp_ref[0] + dp_ref[1] + 1.0
    dinv = lax.rsqrt(deg)
    dinv_ref[...] = dinv
    y_ref[...] = dinv * x_ref[...]


def _k4_body(ap_ref, y_ref, dinv_ref, wb_ref, out_ref):
    s = ap_ref[0] + ap_ref[1] + y_ref[...]
    out0 = dinv_ref[...] * s
    out_ref[0] = out0 * wb_ref[0] + wb_ref[2]
    out_ref[1] = out0 * wb_ref[1] + wb_ref[3]


_R, _C = 800, 125


def kernel(x, edge_index, W, b):
    ei32 = edge_index.astype(jnp.int32)
    zeros = jnp.zeros((N_NODES,), jnp.float32)
    ones = jnp.ones((CHUNK_EDGES,), jnp.float32)

    deg_part = _k1_degree(ei32, zeros, ones)

    x2 = x.reshape(_R, _C)
    y2, dinv2 = pl.pallas_call(
        _k2_body,
        out_shape=[
            jax.ShapeDtypeStruct((_R, _C), jnp.float32),
            jax.ShapeDtypeStruct((_R, _C), jnp.float32),
        ],
    )(deg_part.reshape(NC, _R, _C), x2)

    acc_part = _k3_aggregate(ei32, y2.reshape(N_NODES), zeros)

    wb = jnp.concatenate([W[0], b]).astype(jnp.float32)
    out2 = pl.pallas_call(
        _k4_body,
        out_shape=jax.ShapeDtypeStruct((NC, _R, _C), jnp.float32),
        in_specs=[
            pl.BlockSpec(memory_space=pltpu.VMEM),
            pl.BlockSpec(memory_space=pltpu.VMEM),
            pl.BlockSpec(memory_space=pltpu.VMEM),
            pl.BlockSpec(memory_space=pltpu.SMEM),
        ],
        out_specs=pl.BlockSpec(memory_space=pltpu.VMEM),
    )(acc_part.reshape(NC, _R, _C), y2, dinv2, wb)

    return out2.reshape(NC, N_NODES).T


# speedup vs baseline: 447.2039x; 1.1925x over previous
import functools

import jax
import jax.numpy as jnp
from jax import lax
from jax.experimental import pallas as pl
from jax.experimental.pallas import tpu as pltpu
from jax.experimental.pallas import tpu_sc as plsc

N_NODES = 100000
N_EDGES = 6400000
CHUNK = 2000
NC = 2
NS = 16
NW = NC * NS
NITER = N_EDGES // (CHUNK * NW)

_MESH = plsc.VectorSubcoreMesh(
    core_axis_name="c", subcore_axis_name="s", num_cores=NC, num_subcores=NS
)


def _worker_id():
    c = lax.axis_index("c")
    s = lax.axis_index("s")
    return c, s, s * NC + c


@functools.partial(
    pl.kernel,
    out_type=jax.ShapeDtypeStruct((NC, N_NODES), jnp.float32),
    mesh=_MESH,
    compiler_params=pltpu.CompilerParams(needs_layout_passes=False),
    scratch_types=[
        pltpu.VMEM((CHUNK,), jnp.int32),
        pltpu.VMEM((CHUNK,), jnp.int32),
        pltpu.VMEM((CHUNK,), jnp.float32),
        pltpu.VMEM_SHARED((N_NODES,), jnp.float32),
        pltpu.SemaphoreType.DMA,
        pltpu.SemaphoreType.DMA,
        pltpu.SemaphoreType.DMA,
        pltpu.SemaphoreType.DMA,
    ],
)
def _k1_degree(dst_hbm, zeros_hbm, ones_hbm, deg_out,
               dst0, dst1, ones_v, degacc, in0, in1, sc0, sc1):
    c, s, w = _worker_id()
    dst = (dst0, dst1)
    in_sem = (in0, in1)
    sc_sem = (sc0, sc1)

    @pl.when(s == 0)
    def _():
        pltpu.sync_copy(zeros_hbm, degacc)

    pltpu.sync_copy(ones_hbm, ones_v)
    plsc.subcore_barrier()

    def _in(k, b):
        pltpu.async_copy(
            dst_hbm.at[pl.ds((w + NW * k) * CHUNK, CHUNK)], dst[b], in_sem[b]
        )

    def _wait_in(b):
        pltpu.make_async_copy(
            dst_hbm.at[pl.ds(0, CHUNK)], dst[b], in_sem[b]
        ).wait()

    def _wait_sc(b):
        pltpu.make_async_copy(ones_v, degacc.at[dst[b]], sc_sem[b]).wait()

    _in(0, 0)

    def body(i, carry):
        for b in (0, 1):
            k = 2 * i + b
            _wait_in(b)
            pltpu.async_copy(ones_v, degacc.at[dst[b]], sc_sem[b], add=True)
            if b == 0:
                @pl.when(i > 0)
                def _():
                    _wait_sc(1)
                _in(k + 1, 1)
            else:
                _wait_sc(0)

                @pl.when(i < NITER // 2 - 1)
                def _():
                    _in(k + 1, 0)

        return carry

    lax.fori_loop(0, NITER // 2, body, None)
    _wait_sc(1)
    plsc.subcore_barrier()

    @pl.when(s == 0)
    def _():
        pltpu.sync_copy(degacc, deg_out.at[c])


@functools.partial(
    pl.kernel,
    out_type=jax.ShapeDtypeStruct((NC, N_NODES), jnp.float32),
    mesh=_MESH,
    compiler_params=pltpu.CompilerParams(needs_layout_passes=False),
    scratch_types=[
        pltpu.VMEM((N_NODES,), jnp.float32),
        pltpu.VMEM((CHUNK,), jnp.int32),
        pltpu.VMEM((CHUNK,), jnp.int32),
        pltpu.VMEM((CHUNK,), jnp.int32),
        pltpu.VMEM((CHUNK,), jnp.int32),
        pltpu.VMEM((CHUNK,), jnp.float32),
        pltpu.VMEM((CHUNK,), jnp.float32),
        pltpu.VMEM_SHARED((N_NODES,), jnp.float32),
        pltpu.SemaphoreType.DMA,
        pltpu.SemaphoreType.DMA,
        pltpu.SemaphoreType.DMA,
        pltpu.SemaphoreType.DMA,
    ],
)
def _k3_aggregate(src_hbm, dst_hbm, y_hbm, zeros_hbm, acc_out,
                  ybuf, src0, src1, dst0, dst1, msg0, msg1, sacc,
                  in0, in1, sc0, sc1):
    c, s, w = _worker_id()
    src = (src0, src1)
    dst = (dst0, dst1)
    msg = (msg0, msg1)
    in_sem = (in0, in1)
    sc_sem = (sc0, sc1)

    @pl.when(s == 0)
    def _():
        pltpu.sync_copy(zeros_hbm, sacc)

    pltpu.sync_copy(y_hbm, ybuf)
    plsc.subcore_barrier()

    def _in(k, b):
        base = (w + NW * k) * CHUNK
        pltpu.async_copy(src_hbm.at[pl.ds(base, CHUNK)], src[b], in_sem[b])
        pltpu.async_copy(dst_hbm.at[pl.ds(base, CHUNK)], dst[b], in_sem[b])

    def _wait_in(b):
        pltpu.make_async_copy(
            src_hbm.at[pl.ds(0, CHUNK)], src[b], in_sem[b]
        ).wait()
        pltpu.make_async_copy(
            dst_hbm.at[pl.ds(0, CHUNK)], dst[b], in_sem[b]
        ).wait()

    def _wait_sc(b):
        pltpu.make_async_copy(msg[b], sacc.at[dst[b]], sc_sem[b]).wait()

    _in(0, 0)

    def body(i, carry):
        for b in (0, 1):
            k = 2 * i + b
            _wait_in(b)
            for t in range(CHUNK // 16):
                idx16 = src[b][pl.ds(t * 16, 16)]
                msg[b][pl.ds(t * 16, 16)] = plsc.load_gather(ybuf, [idx16])
            pltpu.async_copy(msg[b], sacc.at[dst[b]], sc_sem[b], add=True)
            if b == 0:
                @pl.when(i > 0)
                def _():
                    _wait_sc(1)
                _in(k + 1, 1)
            else:
                _wait_sc(0)

                @pl.when(i < NITER // 2 - 1)
                def _():
                    _in(k + 1, 0)

        return carry

    lax.fori_loop(0, NITER // 2, body, None)
    _wait_sc(1)
    plsc.subcore_barrier()

    @pl.when(s == 0)
    def _():
        pltpu.sync_copy(sacc, acc_out.at[c])


def _k2_body(dp_ref, x_ref, y_ref, dinv_ref):
    deg = dp_ref[0] + dp_ref[1] + 1.0
    dinv = lax.rsqrt(deg)
    dinv_ref[...] = dinv
    y_ref[...] = dinv * x_ref[...]


def _k4_body(ap_ref, y_ref, dinv_ref, wb_ref, out_ref):
    s = ap_ref[0] + ap_ref[1] + y_ref[...]
    out0 = dinv_ref[...] * s
    out_ref[0] = out0 * wb_ref[0] + wb_ref[2]
    out_ref[1] = out0 * wb_ref[1] + wb_ref[3]


_R, _C = 800, 125


def kernel(x, edge_index, W, b):
    ei32 = edge_index.astype(jnp.int32)
    zeros = jnp.zeros((N_NODES,), jnp.float32)
    ones = jnp.ones((CHUNK,), jnp.float32)

    deg_part = _k1_degree(ei32[1], zeros, ones)

    x2 = x.reshape(_R, _C)
    y2, dinv2 = pl.pallas_call(
        _k2_body,
        out_shape=[
            jax.ShapeDtypeStruct((_R, _C), jnp.float32),
            jax.ShapeDtypeStruct((_R, _C), jnp.float32),
        ],
    )(deg_part.reshape(NC, _R, _C), x2)

    acc_part = _k3_aggregate(ei32[0], ei32[1], y2.reshape(N_NODES), zeros)

    wb = jnp.concatenate([W[0], b]).astype(jnp.float32)
    out2 = pl.pallas_call(
        _k4_body,
        out_shape=jax.ShapeDtypeStruct((NC, _R, _C), jnp.float32),
        in_specs=[
            pl.BlockSpec(memory_space=pltpu.VMEM),
            pl.BlockSpec(memory_space=pltpu.VMEM),
            pl.BlockSpec(memory_space=pltpu.VMEM),
            pl.BlockSpec(memory_space=pltpu.SMEM),
        ],
        out_specs=pl.BlockSpec(memory_space=pltpu.VMEM),
    )(acc_part.reshape(NC, _R, _C), y2, dinv2, wb)

    return out2.reshape(NC, N_NODES).T


# speedup vs baseline: 489.3636x; 1.0943x over previous
import functools

import jax
import jax.numpy as jnp
from jax import lax
from jax.experimental import pallas as pl
from jax.experimental.pallas import tpu as pltpu
from jax.experimental.pallas import tpu_sc as plsc

N_NODES = 100000
N_EDGES = 6400000
NC = 2
NS = 16
NW = NC * NS
CHUNK1 = 10000
NITER1 = N_EDGES // (CHUNK1 * NW)
CHUNK = 4000
NITER = N_EDGES // (CHUNK * NW)

_MESH = plsc.VectorSubcoreMesh(
    core_axis_name="c", subcore_axis_name="s", num_cores=NC, num_subcores=NS
)


def _worker_id():
    c = lax.axis_index("c")
    s = lax.axis_index("s")
    return c, s, s * NC + c


@functools.partial(
    pl.kernel,
    out_type=jax.ShapeDtypeStruct((NC, N_NODES), jnp.float32),
    mesh=_MESH,
    compiler_params=pltpu.CompilerParams(needs_layout_passes=False),
    scratch_types=[
        pltpu.VMEM((CHUNK1,), jnp.int32),
        pltpu.VMEM((CHUNK1,), jnp.int32),
        pltpu.VMEM((CHUNK1,), jnp.float32),
        pltpu.VMEM_SHARED((N_NODES,), jnp.float32),
        pltpu.SemaphoreType.DMA,
        pltpu.SemaphoreType.DMA,
        pltpu.SemaphoreType.DMA,
        pltpu.SemaphoreType.DMA,
    ],
)
def _k1_degree(dst_hbm, zeros_hbm, ones_hbm, deg_out,
               dst0, dst1, ones_v, degacc, in0, in1, sc0, sc1):
    c, s, w = _worker_id()
    dst = (dst0, dst1)
    in_sem = (in0, in1)
    sc_sem = (sc0, sc1)

    @pl.when(s == 0)
    def _():
        pltpu.sync_copy(zeros_hbm, degacc)

    pltpu.sync_copy(ones_hbm, ones_v)
    plsc.subcore_barrier()

    def _in(k, b):
        pltpu.async_copy(
            dst_hbm.at[pl.ds((w + NW * k) * CHUNK1, CHUNK1)], dst[b], in_sem[b]
        )

    def _wait_in(b):
        pltpu.make_async_copy(
            dst_hbm.at[pl.ds(0, CHUNK1)], dst[b], in_sem[b]
        ).wait()

    def _wait_sc(b):
        pltpu.make_async_copy(ones_v, degacc.at[dst[b]], sc_sem[b]).wait()

    _in(0, 0)

    def body(i, carry):
        for b in (0, 1):
            k = 2 * i + b
            _wait_in(b)
            pltpu.async_copy(ones_v, degacc.at[dst[b]], sc_sem[b], add=True)
            if b == 0:
                @pl.when(i > 0)
                def _():
                    _wait_sc(1)
                _in(k + 1, 1)
            else:
                _wait_sc(0)

                @pl.when(i < NITER1 // 2 - 1)
                def _():
                    _in(k + 1, 0)

        return carry

    lax.fori_loop(0, NITER1 // 2, body, None)
    _wait_sc(1)
    plsc.subcore_barrier()

    @pl.when(s == 0)
    def _():
        pltpu.sync_copy(degacc, deg_out.at[c])


@functools.partial(
    pl.kernel,
    out_type=jax.ShapeDtypeStruct((NC, N_NODES), jnp.float32),
    mesh=_MESH,
    compiler_params=pltpu.CompilerParams(needs_layout_passes=False),
    scratch_types=[
        pltpu.VMEM((N_NODES,), jnp.float32),
        pltpu.VMEM((CHUNK,), jnp.int32),
        pltpu.VMEM((CHUNK,), jnp.int32),
        pltpu.VMEM((CHUNK,), jnp.int32),
        pltpu.VMEM((CHUNK,), jnp.int32),
        pltpu.VMEM((CHUNK,), jnp.float32),
        pltpu.VMEM((CHUNK,), jnp.float32),
        pltpu.VMEM_SHARED((N_NODES,), jnp.float32),
        pltpu.SemaphoreType.DMA,
        pltpu.SemaphoreType.DMA,
        pltpu.SemaphoreType.DMA,
        pltpu.SemaphoreType.DMA,
    ],
)
def _k3_aggregate(src_hbm, dst_hbm, y_hbm, zeros_hbm, acc_out,
                  ybuf, src0, src1, dst0, dst1, msg0, msg1, sacc,
                  in0, in1, sc0, sc1):
    c, s, w = _worker_id()
    src = (src0, src1)
    dst = (dst0, dst1)
    msg = (msg0, msg1)
    in_sem = (in0, in1)
    sc_sem = (sc0, sc1)

    @pl.when(s == 0)
    def _():
        pltpu.sync_copy(zeros_hbm, sacc)

    pltpu.sync_copy(y_hbm, ybuf)
    plsc.subcore_barrier()

    def _in(k, b):
        base = (w + NW * k) * CHUNK
        pltpu.async_copy(src_hbm.at[pl.ds(base, CHUNK)], src[b], in_sem[b])
        pltpu.async_copy(dst_hbm.at[pl.ds(base, CHUNK)], dst[b], in_sem[b])

    def _wait_in(b):
        pltpu.make_async_copy(
            src_hbm.at[pl.ds(0, CHUNK)], src[b], in_sem[b]
        ).wait()
        pltpu.make_async_copy(
            dst_hbm.at[pl.ds(0, CHUNK)], dst[b], in_sem[b]
        ).wait()

    def _wait_sc(b):
        pltpu.make_async_copy(msg[b], sacc.at[dst[b]], sc_sem[b]).wait()

    _in(0, 0)

    def body(i, carry):
        for b in (0, 1):
            k = 2 * i + b
            _wait_in(b)
            for t in range(CHUNK // 16):
                idx16 = src[b][pl.ds(t * 16, 16)]
                msg[b][pl.ds(t * 16, 16)] = plsc.load_gather(ybuf, [idx16])
            pltpu.async_copy(msg[b], sacc.at[dst[b]], sc_sem[b], add=True)
            if b == 0:
                @pl.when(i > 0)
                def _():
                    _wait_sc(1)
                _in(k + 1, 1)
            else:
                _wait_sc(0)

                @pl.when(i < NITER // 2 - 1)
                def _():
                    _in(k + 1, 0)

        return carry

    lax.fori_loop(0, NITER // 2, body, None)
    _wait_sc(1)
    plsc.subcore_barrier()

    @pl.when(s == 0)
    def _():
        pltpu.sync_copy(sacc, acc_out.at[c])


def _k2_body(dp_ref, x_ref, y_ref, dinv_ref):
    deg = dp_ref[0] + dp_ref[1] + 1.0
    dinv = lax.rsqrt(deg)
    dinv_ref[...] = dinv
    y_ref[...] = dinv * x_ref[...]


def _k4_body(ap_ref, y_ref, dinv_ref, wb_ref, out_ref):
    s = ap_ref[0] + ap_ref[1] + y_ref[...]
    out0 = dinv_ref[...] * s
    out_ref[0] = out0 * wb_ref[0] + wb_ref[2]
    out_ref[1] = out0 * wb_ref[1] + wb_ref[3]


_R, _C = 800, 125


def kernel(x, edge_index, W, b):
    ei32 = edge_index.astype(jnp.int32)
    zeros = jnp.zeros((N_NODES,), jnp.float32)
    ones = jnp.ones((CHUNK1,), jnp.float32)

    deg_part = _k1_degree(ei32[1], zeros, ones)

    x2 = x.reshape(_R, _C)
    y2, dinv2 = pl.pallas_call(
        _k2_body,
        out_shape=[
            jax.ShapeDtypeStruct((_R, _C), jnp.float32),
            jax.ShapeDtypeStruct((_R, _C), jnp.float32),
        ],
    )(deg_part.reshape(NC, _R, _C), x2)

    acc_part = _k3_aggregate(ei32[0], ei32[1], y2.reshape(N_NODES), zeros)

    wb = jnp.concatenate([W[0], b]).astype(jnp.float32)
    out2 = pl.pallas_call(
        _k4_body,
        out_shape=jax.ShapeDtypeStruct((NC, _R, _C), jnp.float32),
        in_specs=[
            pl.BlockSpec(memory_space=pltpu.VMEM),
            pl.BlockSpec(memory_space=pltpu.VMEM),
            pl.BlockSpec(memory_space=pltpu.VMEM),
            pl.BlockSpec(memory_space=pltpu.SMEM),
        ],
        out_specs=pl.BlockSpec(memory_space=pltpu.VMEM),
    )(acc_part.reshape(NC, _R, _C), y2, dinv2, wb)

    return out2.reshape(NC, N_NODES).T
